# baseline (speedup 1.0000x reference)
import functools

import jax
import jax.numpy as jnp
from jax import lax
from jax.experimental import pallas as pl
from jax.experimental.pallas import tpu as pltpu
from jax.experimental.pallas import tpu_sc as plsc

NUM_COND = 1_000_000
DIM = 16
BATCH = 16384
LANES = 128

_info = plsc.get_sparse_core_info()
_NC, _NS = _info.num_cores, _info.num_subcores
_NW = _NC * _NS
_B_PER_W = BATCH // _NW
_GRP = 16
_NGRP = _B_PER_W // _GRP


def _build():
    mesh = plsc.VectorSubcoreMesh(core_axis_name="c", subcore_axis_name="s")

    @functools.partial(
        pl.kernel,
        mesh=mesh,
        out_type=jax.ShapeDtypeStruct((DIM, BATCH), jnp.float32),
        scratch_types=[
            pltpu.VMEM((_B_PER_W,), jnp.int32),
            pltpu.VMEM((_B_PER_W,), jnp.int32),
            pltpu.VMEM((_GRP, DIM, LANES), jnp.float32),
            pltpu.VMEM((DIM, _B_PER_W), jnp.float32),
            pltpu.SemaphoreType.DMA,
        ],
        compiler_params=pltpu.CompilerParams(needs_layout_passes=False),
    )
    def gather_kernel(table_hbm, blk_hbm, lane_hbm, out_hbm,
                      lane_v, blk_v, stage_v, rows_v, sem):
        wid = lax.axis_index("s") * _NC + lax.axis_index("c")
        base = wid * _B_PER_W
        pltpu.sync_copy(lane_hbm.at[wid], lane_v)
        pltpu.sync_copy(blk_hbm.at[wid], blk_v)

        def group(g, carry):
            blkvec = blk_v[pl.ds(g * _GRP, _GRP)]
            copies = []
            for j in range(_GRP):
                blk = pl.multiple_of(blkvec[j], LANES)
                copies.append(pltpu.async_copy(
                    table_hbm.at[:, pl.ds(blk, LANES)], stage_v.at[j], sem))
            for cp in copies:
                cp.wait()
            jrow = lax.iota(jnp.int32, _GRP)
            lanes = lane_v[pl.ds(g * _GRP, _GRP)]
            for d in range(DIM):
                vals = plsc.load_gather(
                    stage_v, [jrow, jnp.full((_GRP,), d, jnp.int32), lanes])
                rows_v[d, pl.ds(g * _GRP, _GRP)] = vals
            return carry

        lax.fori_loop(0, _NGRP, group, 0)
        pltpu.sync_copy(rows_v, out_hbm.at[:, pl.ds(base, _B_PER_W)])

    return gather_kernel


_gather = _build()


def kernel(embeddings, condition_idx):
    idx = condition_idx.astype(jnp.int32)
    blk = (idx & ~(LANES - 1)).reshape(_NW, _B_PER_W)
    lane = (idx & (LANES - 1)).reshape(_NW, _B_PER_W)
    out_t = _gather(embeddings.T, blk, lane)
    return out_t.T


# speedup vs baseline: 1.1478x; 1.1478x over previous
import functools

import jax
import jax.numpy as jnp
from jax import lax
from jax.experimental import pallas as pl
from jax.experimental.pallas import tpu as pltpu
from jax.experimental.pallas import tpu_sc as plsc

NUM_COND = 1_000_000
DIM = 16
BATCH = 16384
LANES = 128

_info = plsc.get_sparse_core_info()
_NC, _NS = _info.num_cores, _info.num_subcores
_NW = _NC * _NS
_B_PER_W = BATCH // _NW
_GRP = 16
_NGRP = _B_PER_W // _GRP


def _build():
    mesh = plsc.VectorSubcoreMesh(core_axis_name="c", subcore_axis_name="s")

    @functools.partial(
        pl.kernel,
        mesh=mesh,
        out_type=jax.ShapeDtypeStruct((DIM, BATCH), jnp.float32),
        scratch_types=[
            pltpu.VMEM((_B_PER_W,), jnp.int32),
            pltpu.VMEM((_B_PER_W,), jnp.int32),
            pltpu.VMEM((2, _GRP, DIM, LANES), jnp.float32),
            pltpu.VMEM((DIM, _B_PER_W), jnp.float32),
            pltpu.SemaphoreType.DMA,
            pltpu.SemaphoreType.DMA,
        ],
        compiler_params=pltpu.CompilerParams(needs_layout_passes=False),
    )
    def gather_kernel(table_hbm, blk_hbm, lane_hbm, out_hbm,
                      lane_v, blk_v, stage_v, rows_v, semA, semB):
        wid = lax.axis_index("s") * _NC + lax.axis_index("c")
        base = wid * _B_PER_W
        pltpu.sync_copy(lane_hbm.at[wid], lane_v)
        pltpu.sync_copy(blk_hbm.at[wid], blk_v)
        sems = (semA, semB)

        def fire(g, slot, sem):
            blkvec = blk_v[pl.ds(g * _GRP, _GRP)]
            for j in range(_GRP):
                blk = pl.multiple_of(blkvec[j], LANES)
                pltpu.async_copy(
                    table_hbm.at[:, pl.ds(blk, LANES)],
                    stage_v.at[slot, j], sem)

        def drain_extract(g, slot, sem):
            for j in range(_GRP):
                pltpu.make_async_copy(
                    table_hbm.at[:, pl.ds(0, LANES)],
                    stage_v.at[slot, j], sem).wait()
            jrow = lax.iota(jnp.int32, _GRP)
            srow = jnp.full((_GRP,), slot, jnp.int32)
            lanes = lane_v[pl.ds(g * _GRP, _GRP)]
            for d in range(DIM):
                vals = plsc.load_gather(
                    stage_v, [srow, jrow, jnp.full((_GRP,), d, jnp.int32),
                              lanes])
                rows_v[d, pl.ds(g * _GRP, _GRP)] = vals

        fire(0, 0, semA)
        fire(1, 1, semB)

        def step(h, carry):
            for slot in range(2):
                g = 2 * h + slot
                drain_extract(g, slot, sems[slot])
                fire(g + 2, slot, sems[slot])
            return carry

        lax.fori_loop(0, _NGRP // 2 - 1, step, 0)
        for slot in range(2):
            drain_extract(_NGRP - 2 + slot, slot, sems[slot])

        pltpu.sync_copy(rows_v, out_hbm.at[:, pl.ds(base, _B_PER_W)])

    return gather_kernel


_gather = _build()


def kernel(embeddings, condition_idx):
    idx = condition_idx.astype(jnp.int32)
    blk = (idx & ~(LANES - 1)).reshape(_NW, _B_PER_W)
    lane = (idx & (LANES - 1)).reshape(_NW, _B_PER_W)
    out_t = _gather(embeddings.T, blk, lane)
    return out_t.T


# speedup vs baseline: 1.2647x; 1.1018x over previous
import functools

import jax
import jax.numpy as jnp
from jax import lax
from jax.experimental import pallas as pl
from jax.experimental.pallas import tpu as pltpu
from jax.experimental.pallas import tpu_sc as plsc

NUM_COND = 1_000_000
DIM = 16
BATCH = 16384
LANES = 128

_info = plsc.get_sparse_core_info()
_NC, _NS = _info.num_cores, _info.num_subcores
_NW = _NC * _NS
_B_PER_W = BATCH // _NW
_GRP = 16
_NGRP = _B_PER_W // _GRP
_NSLOT = 3


def _build():
    mesh = plsc.VectorSubcoreMesh(core_axis_name="c", subcore_axis_name="s")

    @functools.partial(
        pl.kernel,
        mesh=mesh,
        out_type=jax.ShapeDtypeStruct((DIM, BATCH), jnp.float32),
        scratch_types=[
            pltpu.VMEM((_B_PER_W,), jnp.int32),
            pltpu.VMEM((_NSLOT, DIM, _GRP * LANES), jnp.float32),
            pltpu.VMEM((DIM, _B_PER_W), jnp.float32),
            pltpu.SemaphoreType.DMA,
            pltpu.SemaphoreType.DMA,
            pltpu.SemaphoreType.DMA,
        ],
        compiler_params=pltpu.CompilerParams(needs_layout_passes=False),
    )
    def gather_kernel(table_hbm, idx_hbm, out_hbm,
                      idx_v, stage_v, rows_v, semA, semB, semC):
        wid = lax.axis_index("s") * _NC + lax.axis_index("c")
        base = wid * _B_PER_W
        pltpu.sync_copy(idx_hbm.at[wid], idx_v)
        sems = (semA, semB, semC)

        def fire(g, slot, sem):
            idxvec = idx_v[pl.ds(g * _GRP, _GRP)]
            blkvec = idxvec & jnp.int32(~(LANES - 1))
            for j in range(_GRP):
                blk = pl.multiple_of(blkvec[j], LANES)
                pltpu.async_copy(
                    table_hbm.at[:, pl.ds(blk, LANES)],
                    stage_v.at[slot, :, pl.ds(j * LANES, LANES)], sem)

        def drain_extract(g, slot, sem):
            pltpu.make_async_copy(
                table_hbm.at[:, pl.ds(0, _GRP * LANES)],
                stage_v.at[slot], sem).wait()
            lanes = idx_v[pl.ds(g * _GRP, _GRP)] & jnp.int32(LANES - 1)
            cols = lax.iota(jnp.int32, _GRP) * LANES + lanes
            srow = jnp.full((_GRP,), slot, jnp.int32)
            for d in range(DIM):
                vals = plsc.load_gather(
                    stage_v, [srow, jnp.full((_GRP,), d, jnp.int32), cols])
                rows_v[d, pl.ds(g * _GRP, _GRP)] = vals

        for g in range(_NSLOT):
            fire(g, g, sems[g])

        def step(h, carry):
            for s in range(_NSLOT):
                g = _NSLOT * h + s
                drain_extract(g, s, sems[s])
                fire(g + _NSLOT, s, sems[s])
            return carry

        _NFULL = (_NGRP - _NSLOT) // _NSLOT
        lax.fori_loop(0, _NFULL, step, 0)
        for g in range(_NFULL * _NSLOT, _NGRP):
            s = g % _NSLOT
            drain_extract(g, s, sems[s])
            if g + _NSLOT < _NGRP:
                fire(g + _NSLOT, s, sems[s])

        pltpu.sync_copy(rows_v, out_hbm.at[:, pl.ds(base, _B_PER_W)])

    return gather_kernel


_gather = _build()


def kernel(embeddings, condition_idx):
    idx = condition_idx.astype(jnp.int32).reshape(_NW, _B_PER_W)
    out_t = _gather(embeddings.T, idx)
    return out_t.T
